# baseline (speedup 1.0000x reference)
import functools

import jax
import jax.numpy as jnp
from jax import lax
from jax.experimental import pallas as pl
from jax.experimental.pallas import tpu as pltpu
from jax.experimental.pallas import tpu_sc as plsc

NPTS = 320000
NSEG = 10000
NSEG_PAD = 10016
CH = 128
NVR = CH // 16

NW = 32
ROWS_W = NPTS // NW
CHUNK = 200
NCHUNKS = ROWS_W // CHUNK
KB = 16
FILL = 64
NFILL = NSEG_PAD // FILL + 1
PAD_LO = 8
PAD_HI = 16
IDS_W = ROWS_W + PAD_LO + PAD_HI

_mesh = plsc.VectorSubcoreMesh(core_axis_name="c", subcore_axis_name="s")


@functools.partial(
    pl.kernel,
    mesh=_mesh,
    compiler_params=pltpu.CompilerParams(use_tc_tiling_on_sc=False),
    out_type=(
        jax.ShapeDtypeStruct((NSEG_PAD, CH), jnp.float32),
        jax.ShapeDtypeStruct((64 * CH,), jnp.float32),
        jax.ShapeDtypeStruct((64 * 16,), jnp.int32),
    ),
    scratch_types=[
        pltpu.VMEM((IDS_W,), jnp.int32),
        pltpu.VMEM((CHUNK * CH,), jnp.float32),
        pltpu.VMEM((CHUNK * CH,), jnp.float32),
        pltpu.VMEM((FILL, CH), jnp.float32),
        pltpu.VMEM((KB, CH), jnp.float32),
        pltpu.VMEM((KB,), jnp.int32),
        pltpu.VMEM((2 * CH,), jnp.float32),
        pltpu.VMEM((2 * 16,), jnp.int32),
        pltpu.SemaphoreType.DMA,
        pltpu.SemaphoreType.DMA,
        pltpu.SemaphoreType.DMA,
    ],
)
def _segmin_kernel(ids_hbm, pts_hbm, inf_hbm, base_hbm, part_hbm, pid_hbm,
                   ids_v, buf0, buf1, inf_v, stg, idx_v, pstage, pidst,
                   sem0, sem1, semf):
    w = lax.axis_index("s") * 2 + lax.axis_index("c")
    row0 = w * ROWS_W
    fmax16 = jnp.full((16,), 3.4028235e38, dtype=jnp.float32)
    iota16 = lax.iota(jnp.int32, 16)
    dummy = jnp.int32(NSEG_PAD - 1)

    pltpu.sync_copy(ids_hbm.at[pl.ds(row0, IDS_W)], ids_v)
    pltpu.async_copy(inf_hbm, inf_v, semf).wait()

    head = ids_v[pl.ds(0, 16)]
    lo = head[PAD_LO - 1] + 1
    first_id = head[PAD_LO]
    my_last = ids_v[pl.ds(PAD_LO + ROWS_W - 1, 16)][0]
    hi_end = jnp.where(w == NW - 1, jnp.int32(NSEG_PAD - 1), my_last)

    rem0 = jnp.maximum(hi_end + 1 - lo, 0)
    n64 = rem0 // FILL

    def fill64(j, _):
        @pl.when(j < n64)
        def _():
            pltpu.async_copy(inf_v, base_hbm.at[pl.ds(lo + j * FILL, FILL)],
                             semf).wait()
        return 0

    lax.fori_loop(0, NFILL, fill64, 0)
    start8 = lo + n64 * FILL
    n8 = jnp.maximum(hi_end + 1 - start8, 0) // 8

    def fill8(j, _):
        @pl.when(j < n8)
        def _():
            pltpu.async_copy(inf_v.at[pl.ds(0, 8)],
                             base_hbm.at[pl.ds(start8 + j * 8, 8)],
                             semf).wait()
        return 0

    lax.fori_loop(0, 8, fill8, 0)
    start1 = start8 + n8 * 8

    def fill1(j, _):
        @pl.when(start1 + j <= hi_end)
        def _():
            pltpu.async_copy(inf_v.at[pl.ds(0, 1)],
                             base_hbm.at[pl.ds(start1 + j, 1)],
                             semf).wait()
        return 0

    lax.fori_loop(0, 8, fill1, 0)

    pltpu.async_copy(pts_hbm.at[pl.ds(row0 * CH, CHUNK * CH)], buf0, sem0)

    def do_rows(chunk_idx, buf, carry):
        def row_body(r, c2):
            cur_id, segcnt = c2[0], c2[1]
            accs = c2[2:2 + NVR]
            idxs = c2[2 + NVR:]
            rid = ids_v[pl.ds(PAD_LO + chunk_idx * CHUNK + r, 16)][0]
            ne = jnp.minimum(jnp.abs(rid - cur_id), 1)
            is_batch = ne * jnp.minimum(segcnt, 1)
            slot = (segcnt - 1) & (KB - 1)

            m = (jnp.maximum(1 - jnp.abs(iota16 - slot), 0) * is_batch)
            new_idx = idxs[0] * (1 - m) + cur_id * m

            @pl.when(ne == 1)
            def _():
                @pl.when(segcnt == 0)
                def _():
                    for k in range(NVR):
                        pstage[pl.ds(k * 16, 16)] = accs[k]
                    pidst[pl.ds(0, 16)] = (
                        jnp.broadcast_to(cur_id, (16,)).astype(jnp.int32))

                @pl.when(segcnt > 0)
                def _():
                    row = stg.at[slot]
                    for k in range(NVR):
                        row[pl.ds(k * 16, 16)] = accs[k]

                    @pl.when(slot == KB - 1)
                    def _():
                        idx_v[pl.ds(0, 16)] = new_idx
                        pltpu.async_copy(stg, base_hbm.at[idx_v], semf).wait()

            bf16 = jnp.broadcast_to(ne.astype(jnp.float32) * 3.4028235e38,
                                    (16,))
            new_accs = tuple(
                jnp.minimum(accs[k] + bf16,
                            buf[pl.ds(r * CH + k * 16, 16)])
                for k in range(NVR))
            return (rid, segcnt + ne) + new_accs + (new_idx,)

        return lax.fori_loop(0, CHUNK, row_body, carry)

    carry = (first_id, jnp.int32(0)) + (fmax16,) * NVR + (
        jnp.broadcast_to(dummy, (16,)).astype(jnp.int32),)

    def outer(g, carry):
        base_row = row0 + 2 * g * CHUNK
        pltpu.async_copy(pts_hbm.at[pl.ds((base_row + CHUNK) * CH, CHUNK * CH)],
                         buf1, sem1)
        pltpu.make_async_copy(pts_hbm.at[pl.ds(base_row * CH, CHUNK * CH)],
                              buf0, sem0).wait()
        carry = do_rows(2 * g, buf0, carry)

        @pl.when(g < NCHUNKS // 2 - 1)
        def _():
            pltpu.async_copy(
                pts_hbm.at[pl.ds((base_row + 2 * CHUNK) * CH, CHUNK * CH)],
                buf0, sem0)

        pltpu.make_async_copy(pts_hbm.at[pl.ds((base_row + CHUNK) * CH, CHUNK * CH)],
                              buf1, sem1).wait()
        carry = do_rows(2 * g + 1, buf1, carry)
        return carry

    carry = lax.fori_loop(0, NCHUNKS // 2, outer, carry)
    cur_id, segcnt = carry[0], carry[1]
    accs = carry[2:2 + NVR]
    idxs = carry[2 + NVR:]

    idx_v[pl.ds(0, 16)] = idxs[0]
    pltpu.async_copy(stg, base_hbm.at[idx_v], semf).wait()

    for k in range(NVR):
        pstage[pl.ds(CH + k * 16, 16)] = accs[k]
    pidst[pl.ds(16, 16)] = jnp.broadcast_to(cur_id, (16,)).astype(jnp.int32)

    @pl.when(segcnt == 0)
    def _():
        for k in range(NVR):
            pstage[pl.ds(k * 16, 16)] = accs[k]
        pidst[pl.ds(0, 16)] = jnp.broadcast_to(cur_id, (16,)).astype(jnp.int32)

    pltpu.sync_copy(pstage, part_hbm.at[pl.ds(2 * w * CH, 2 * CH)])
    pltpu.sync_copy(pidst, pid_hbm.at[pl.ds(2 * w * 16, 2 * 16)])


BLK = 1000
NBLK = NSEG // BLK


def _mlp_body(base_ref, part_ref, pidr_ref, pidc_ref,
              w1t_ref, b1_ref, w2t_ref, b2_ref, o_ref):
    i = pl.program_id(0)
    comb = part_ref[...]
    prf = pidr_ref[...].astype(jnp.float32)
    pcf = pidc_ref[...].astype(jnp.float32)
    big = jnp.float32(1e30)

    for s in (1, 2, 4, 8, 16, 32):
        pc_dn = jnp.concatenate(
            [pcf[s:], jnp.full((s, 1), -7.0, jnp.float32)], axis=0)
        cb_dn = jnp.concatenate(
            [comb[s:], jnp.zeros((s, CH), jnp.float32)], axis=0)
        comb = jnp.minimum(comb, cb_dn + jnp.abs(pcf - pc_dn) * big)
        pc_up = jnp.concatenate(
            [jnp.full((s, 1), -8.0, jnp.float32), pcf[:-s]], axis=0)
        cb_up = jnp.concatenate(
            [jnp.zeros((s, CH), jnp.float32), comb[:-s]], axis=0)
        comb = jnp.minimum(comb, cb_up + jnp.abs(pcf - pc_up) * big)

    pr_prev = jnp.concatenate(
        [jnp.full((1, 1), -9.0, jnp.float32), prf[:, :-1]], axis=1)
    neq = jnp.minimum(jnp.abs(prf - pr_prev), 1.0)
    pid_sel = prf * neq + (neq - 1.0)

    rid = (i * BLK + lax.broadcasted_iota(jnp.int32, (BLK, 1), 0)
           ).astype(jnp.float32)
    mf = jnp.maximum(1.0 - jnp.abs(rid - pid_sel), 0.0)
    ovl = jnp.dot(mf, comb, preferred_element_type=jnp.float32)
    msum = jnp.minimum(jnp.sum(mf, axis=1, keepdims=True), 1.0)
    basec = jnp.minimum(base_ref[...], big)
    feat = ovl * msum + basec * (1.0 - msum)

    h = jnp.maximum(
        jnp.dot(feat, w1t_ref[...], preferred_element_type=jnp.float32)
        + b1_ref[...], 0.0)
    z = (jnp.dot(h, w2t_ref[...], preferred_element_type=jnp.float32)
         + b2_ref[...])
    o_ref[...] = 1.0 / (1.0 + jnp.exp(-z))


_mlp = pl.pallas_call(
    _mlp_body,
    grid=(NBLK,),
    in_specs=[
        pl.BlockSpec((BLK, CH), lambda i: (i, 0)),
        pl.BlockSpec((64, CH), lambda i: (0, 0)),
        pl.BlockSpec((1, 64), lambda i: (0, 0)),
        pl.BlockSpec((64, 1), lambda i: (0, 0)),
        pl.BlockSpec((CH, 2 * CH), lambda i: (0, 0)),
        pl.BlockSpec((1, 2 * CH), lambda i: (0, 0)),
        pl.BlockSpec((2 * CH, CH), lambda i: (0, 0)),
        pl.BlockSpec((1, CH), lambda i: (0, 0)),
    ],
    out_specs=pl.BlockSpec((BLK, CH), lambda i: (i, 0)),
    out_shape=jax.ShapeDtypeStruct((NSEG, CH), jnp.float32),
)


@jax.jit
def kernel(points, unq_inv, W1, b1, W2, b2):
    ids = unq_inv.astype(jnp.int32)
    ids_ext = jnp.concatenate([
        jnp.full((PAD_LO,), -1, jnp.int32),
        ids,
        jnp.full((PAD_HI,), 2 ** 30, jnp.int32),
    ])
    inf_fill = jnp.full((FILL, CH), jnp.inf, jnp.float32)
    base, part, pid16 = _segmin_kernel(ids_ext, points.reshape(-1), inf_fill)
    pid16 = pid16.reshape(64, 16)
    pid_row = pid16[:, 0].reshape(1, 64)
    pid_col = pid16[:, 0].reshape(64, 1)
    return _mlp(base, part.reshape(64, CH), pid_row, pid_col,
                W1.T, b1.reshape(1, -1), W2.T, b2.reshape(1, -1))


# speedup vs baseline: 1.6005x; 1.6005x over previous
import functools

import jax
import jax.numpy as jnp
from jax import lax
from jax.experimental import pallas as pl
from jax.experimental.pallas import tpu as pltpu
from jax.experimental.pallas import tpu_sc as plsc

NPTS = 320000
NSEG = 10000
NSEG_PAD = 10016
CH = 128
NVR = CH // 16

NW = 32
ROWS_W = NPTS // NW
CHUNK = 200
NCHUNKS = ROWS_W // CHUNK
KB = 16
FILL = 64
NFILL = NSEG_PAD // FILL + 1
PAD_LO = 8
PAD_HI = 16
IDS_W = ROWS_W + PAD_LO + PAD_HI

_mesh = plsc.VectorSubcoreMesh(core_axis_name="c", subcore_axis_name="s")


@functools.partial(
    pl.kernel,
    mesh=_mesh,
    compiler_params=pltpu.CompilerParams(use_tc_tiling_on_sc=False),
    out_type=(
        jax.ShapeDtypeStruct((NSEG_PAD, CH), jnp.float32),
        jax.ShapeDtypeStruct((64 * CH,), jnp.float32),
        jax.ShapeDtypeStruct((64 * 16,), jnp.int32),
    ),
    scratch_types=[
        pltpu.VMEM((IDS_W,), jnp.int32),
        pltpu.VMEM((CHUNK * CH,), jnp.float32),
        pltpu.VMEM((CHUNK * CH,), jnp.float32),
        pltpu.VMEM((FILL, CH), jnp.float32),
        pltpu.VMEM((KB, CH), jnp.float32),
        pltpu.VMEM((KB,), jnp.int32),
        pltpu.VMEM((CH,), jnp.float32),
        pltpu.VMEM((2 * CH,), jnp.float32),
        pltpu.VMEM((2 * 16,), jnp.int32),
        pltpu.SMEM((8,), jnp.int32),
        pltpu.SemaphoreType.DMA,
        pltpu.SemaphoreType.DMA,
        pltpu.SemaphoreType.DMA,
    ],
)
def _segmin_kernel(ids_hbm, pts_hbm, inf_hbm, base_hbm, part_hbm, pid_hbm,
                   ids_v, buf0, buf1, inf_v, stg, idx_v, accv, pstage, pidst,
                   smc, sem0, sem1, semf):
    w = lax.axis_index("s") * 2 + lax.axis_index("c")
    row0 = w * ROWS_W
    fmax16 = jnp.full((16,), 3.4028235e38, dtype=jnp.float32)
    iota16 = lax.iota(jnp.int32, 16)
    dummy = jnp.int32(NSEG_PAD - 1)

    pltpu.sync_copy(ids_hbm.at[pl.ds(row0, IDS_W)], ids_v)
    pltpu.async_copy(inf_hbm, inf_v, semf).wait()

    head = ids_v[pl.ds(0, 16)]
    lo = head[PAD_LO - 1] + 1
    first_id = head[PAD_LO]
    my_last = ids_v[pl.ds(PAD_LO + ROWS_W - 1, 16)][0]
    hi_end = jnp.where(w == NW - 1, jnp.int32(NSEG_PAD - 1), my_last)

    rem0 = jnp.maximum(hi_end + 1 - lo, 0)
    n64 = rem0 // FILL

    def fill64(j, _):
        @pl.when(j < n64)
        def _():
            pltpu.async_copy(inf_v, base_hbm.at[pl.ds(lo + j * FILL, FILL)],
                             semf).wait()
        return 0

    lax.fori_loop(0, NFILL, fill64, 0)
    start8 = lo + n64 * FILL
    n8 = jnp.maximum(hi_end + 1 - start8, 0) // 8

    def fill8(j, _):
        @pl.when(j < n8)
        def _():
            pltpu.async_copy(inf_v.at[pl.ds(0, 8)],
                             base_hbm.at[pl.ds(start8 + j * 8, 8)],
                             semf).wait()
        return 0

    lax.fori_loop(0, 8, fill8, 0)
    start1 = start8 + n8 * 8

    def fill1(j, _):
        @pl.when(start1 + j <= hi_end)
        def _():
            pltpu.async_copy(inf_v.at[pl.ds(0, 1)],
                             base_hbm.at[pl.ds(start1 + j, 1)],
                             semf).wait()
        return 0

    lax.fori_loop(0, 8, fill1, 0)

    pltpu.async_copy(pts_hbm.at[pl.ds(row0 * CH, CHUNK * CH)], buf0, sem0)
    idx_v[pl.ds(0, 16)] = jnp.broadcast_to(dummy, (16,)).astype(jnp.int32)
    for k in range(NVR):
        accv[pl.ds(k * 16, 16)] = fmax16

    GB = 8

    def do_rows(chunk_idx, buf, carry):
        def grp_body(g, c2):
            cur_id, segcnt = c2
            pos = PAD_LO + chunk_idx * CHUNK + g * GB
            rowbase = g * GB * CH
            idv = ids_v[pl.ds(pos, 16)]
            impure = (jnp.minimum(jnp.abs(idv[0] - cur_id), 1)
                      + jnp.minimum(jnp.abs(idv[GB - 1] - cur_id), 1))
            smc[0] = segcnt

            @pl.when(impure == 0)
            def _():
                for k in range(NVR):
                    c = [buf[pl.ds(rowbase + r * CH + k * 16, 16)]
                         for r in range(GB)]
                    m01 = jnp.minimum(jnp.minimum(c[0], c[1]),
                                      jnp.minimum(c[2], c[3]))
                    m23 = jnp.minimum(jnp.minimum(c[4], c[5]),
                                      jnp.minimum(c[6], c[7]))
                    gmin = jnp.minimum(m01, m23)
                    accv[pl.ds(k * 16, 16)] = (
                        jnp.minimum(accv[pl.ds(k * 16, 16)], gmin))

            @pl.when(impure > 0)
            def _():
                accs = [accv[pl.ds(k * 16, 16)] for k in range(NVR)]
                cur = cur_id
                sc = segcnt
                for r in range(GB):
                    rid = idv[r]
                    ne = jnp.minimum(jnp.abs(rid - cur), 1)
                    is_b = ne * jnp.minimum(sc, 1)
                    slot = (sc - 1) & (KB - 1)

                    @pl.when(ne - is_b == 1)
                    def _(accs=accs, cur=cur):
                        for k in range(NVR):
                            pstage[pl.ds(k * 16, 16)] = accs[k]
                        pidst[pl.ds(0, 16)] = (
                            jnp.broadcast_to(cur, (16,)).astype(jnp.int32))

                    @pl.when(is_b == 1)
                    def _(accs=accs, cur=cur, slot=slot):
                        row = stg.at[slot]
                        for k in range(NVR):
                            row[pl.ds(k * 16, 16)] = accs[k]
                        mm = jnp.maximum(1 - jnp.abs(iota16 - slot), 0)
                        idx_v[pl.ds(0, 16)] = (
                            idx_v[pl.ds(0, 16)] * (1 - mm) + cur * mm)

                        @pl.when(slot == KB - 1)
                        def _():
                            pltpu.async_copy(stg, base_hbm.at[idx_v],
                                             semf).wait()

                    bf16 = jnp.broadcast_to(
                        ne.astype(jnp.float32) * 3.4028235e38, (16,))
                    accs = [jnp.minimum(accs[k] + bf16,
                                        buf[pl.ds(rowbase + r * CH + k * 16,
                                                  16)])
                            for k in range(NVR)]
                    cur = rid
                    sc = sc + ne
                for k in range(NVR):
                    accv[pl.ds(k * 16, 16)] = accs[k]
                smc[0] = sc

            return (idv[GB - 1], smc[0])

        return lax.fori_loop(0, CHUNK // GB, grp_body, carry)

    carry = (first_id, jnp.int32(0))

    def outer(g, carry):
        base_row = row0 + 2 * g * CHUNK
        pltpu.async_copy(pts_hbm.at[pl.ds((base_row + CHUNK) * CH, CHUNK * CH)],
                         buf1, sem1)
        pltpu.make_async_copy(pts_hbm.at[pl.ds(base_row * CH, CHUNK * CH)],
                              buf0, sem0).wait()
        carry = do_rows(2 * g, buf0, carry)

        @pl.when(g < NCHUNKS // 2 - 1)
        def _():
            pltpu.async_copy(
                pts_hbm.at[pl.ds((base_row + 2 * CHUNK) * CH, CHUNK * CH)],
                buf0, sem0)

        pltpu.make_async_copy(pts_hbm.at[pl.ds((base_row + CHUNK) * CH, CHUNK * CH)],
                              buf1, sem1).wait()
        carry = do_rows(2 * g + 1, buf1, carry)
        return carry

    carry = lax.fori_loop(0, NCHUNKS // 2, outer, carry)
    cur_id, segcnt = carry
    accs = [accv[pl.ds(k * 16, 16)] for k in range(NVR)]

    pltpu.async_copy(stg, base_hbm.at[idx_v], semf).wait()

    for k in range(NVR):
        pstage[pl.ds(CH + k * 16, 16)] = accs[k]
    pidst[pl.ds(16, 16)] = jnp.broadcast_to(cur_id, (16,)).astype(jnp.int32)

    @pl.when(segcnt == 0)
    def _():
        for k in range(NVR):
            pstage[pl.ds(k * 16, 16)] = accs[k]
        pidst[pl.ds(0, 16)] = jnp.broadcast_to(cur_id, (16,)).astype(jnp.int32)

    pltpu.sync_copy(pstage, part_hbm.at[pl.ds(2 * w * CH, 2 * CH)])
    pltpu.sync_copy(pidst, pid_hbm.at[pl.ds(2 * w * 16, 2 * 16)])


BLK = 1000
NBLK = NSEG // BLK


def _mlp_body(base_ref, part_ref, pidr_ref, pidc_ref,
              w1t_ref, b1_ref, w2t_ref, b2_ref, o_ref):
    i = pl.program_id(0)
    comb = part_ref[...]
    prf = pidr_ref[...].astype(jnp.float32)
    pcf = pidc_ref[...].astype(jnp.float32)
    big = jnp.float32(1e30)

    for s in (1, 2, 4, 8, 16, 32):
        pc_dn = jnp.concatenate(
            [pcf[s:], jnp.full((s, 1), -7.0, jnp.float32)], axis=0)
        cb_dn = jnp.concatenate(
            [comb[s:], jnp.zeros((s, CH), jnp.float32)], axis=0)
        comb = jnp.minimum(comb, cb_dn + jnp.abs(pcf - pc_dn) * big)
        pc_up = jnp.concatenate(
            [jnp.full((s, 1), -8.0, jnp.float32), pcf[:-s]], axis=0)
        cb_up = jnp.concatenate(
            [jnp.zeros((s, CH), jnp.float32), comb[:-s]], axis=0)
        comb = jnp.minimum(comb, cb_up + jnp.abs(pcf - pc_up) * big)

    pr_prev = jnp.concatenate(
        [jnp.full((1, 1), -9.0, jnp.float32), prf[:, :-1]], axis=1)
    neq = jnp.minimum(jnp.abs(prf - pr_prev), 1.0)
    pid_sel = prf * neq + (neq - 1.0)

    rid = (i * BLK + lax.broadcasted_iota(jnp.int32, (BLK, 1), 0)
           ).astype(jnp.float32)
    mf = jnp.maximum(1.0 - jnp.abs(rid - pid_sel), 0.0)
    ovl = jnp.dot(mf, comb, preferred_element_type=jnp.float32)
    msum = jnp.minimum(jnp.sum(mf, axis=1, keepdims=True), 1.0)
    basec = jnp.minimum(base_ref[...], big)
    feat = ovl * msum + basec * (1.0 - msum)

    h = jnp.maximum(
        jnp.dot(feat, w1t_ref[...], preferred_element_type=jnp.float32)
        + b1_ref[...], 0.0)
    z = (jnp.dot(h, w2t_ref[...], preferred_element_type=jnp.float32)
         + b2_ref[...])
    o_ref[...] = 1.0 / (1.0 + jnp.exp(-z))


_mlp = pl.pallas_call(
    _mlp_body,
    grid=(NBLK,),
    in_specs=[
        pl.BlockSpec((BLK, CH), lambda i: (i, 0)),
        pl.BlockSpec((64, CH), lambda i: (0, 0)),
        pl.BlockSpec((1, 64), lambda i: (0, 0)),
        pl.BlockSpec((64, 1), lambda i: (0, 0)),
        pl.BlockSpec((CH, 2 * CH), lambda i: (0, 0)),
        pl.BlockSpec((1, 2 * CH), lambda i: (0, 0)),
        pl.BlockSpec((2 * CH, CH), lambda i: (0, 0)),
        pl.BlockSpec((1, CH), lambda i: (0, 0)),
    ],
    out_specs=pl.BlockSpec((BLK, CH), lambda i: (i, 0)),
    out_shape=jax.ShapeDtypeStruct((NSEG, CH), jnp.float32),
)


@jax.jit
def kernel(points, unq_inv, W1, b1, W2, b2):
    ids = unq_inv.astype(jnp.int32)
    ids_ext = jnp.concatenate([
        jnp.full((PAD_LO,), -1, jnp.int32),
        ids,
        jnp.full((PAD_HI,), 2 ** 30, jnp.int32),
    ])
    inf_fill = jnp.full((FILL, CH), jnp.inf, jnp.float32)
    base, part, pid16 = _segmin_kernel(ids_ext, points.reshape(-1), inf_fill)
    pid16 = pid16.reshape(64, 16)
    pid_row = pid16[:, 0].reshape(1, 64)
    pid_col = pid16[:, 0].reshape(64, 1)
    return _mlp(base, part.reshape(64, CH), pid_row, pid_col,
                W1.T, b1.reshape(1, -1), W2.T, b2.reshape(1, -1))
